# baseline (speedup 1.0000x reference)
import jax
import jax.numpy as jnp
from jax import lax
from jax.experimental import pallas as pl
from jax.experimental.pallas import tpu as pltpu
from jax.experimental.pallas import tpu_sc as plsc

F_TABLES = 26
B = 4096
V = 100000
D = 64
TOTAL_VALUES = 212992
N_BAGS = F_TABLES * B

NC = 2
NS = 16
NW = NC * NS

BAGS_PER_W = N_BAGS // NW
BAGS_PER_REG = 128
N_REG = BAGS_PER_W // BAGS_PER_REG
ACC_ROWS_PER_W = BAGS_PER_REG + 8
DUMMY_OFF = BAGS_PER_REG
N_SEARCH = 7

K = 1024
N_VECS = K // 16

OFF_LOAD = 3344
OFF_PAD_LEN = (NW - 1) * BAGS_PER_W + OFF_LOAD


def _body(values_hbm, offsets_hbm, tables_hbm, zeros_hbm, out_hbm,
          off_v, vals_v, gidx_v, didx_v, rows_v, zeros_v, acc, sem):
  c = lax.axis_index("c")
  s = lax.axis_index("s")
  wid = c * NS + s
  bag_lo = wid * BAGS_PER_W
  srow = s * ACC_ROWS_PER_W

  lane = lax.iota(jnp.int32, 16)

  pltpu.sync_copy(offsets_hbm.at[pl.ds(bag_lo, OFF_LOAD)], off_v)
  pltpu.sync_copy(zeros_hbm, zeros_v)

  def off_scalar(idx16):
    v = off_v[pl.ds(idx16, 16)]
    return jnp.max(jnp.where(lane == 0, v, jnp.int32(-1)))

  def region_body(h, carry0):
    hbase = h * BAGS_PER_REG
    v_start = off_scalar(hbase)
    v_end = off_scalar(hbase + BAGS_PER_REG)
    f = lax.shift_right_arithmetic(bag_lo + hbase, 12)
    b0 = lax.bitwise_and(bag_lo + hbase, jnp.int32(B - 1))

    pltpu.sync_copy(zeros_v, acc.at[pl.ds(srow, ACC_ROWS_PER_W)])

    c0 = v_start & jnp.int32(~7)
    n_chunks = (v_end - c0 + jnp.int32(K - 1)) // jnp.int32(K)

    def chunk_body(ci, carry):
      base = pl.multiple_of(c0 + ci * jnp.int32(K), 8)
      pltpu.sync_copy(values_hbm.at[pl.ds(base, K)], vals_v)

      def vec_body(g, carry2):
        pos = base + g * 16 + lane
        vals = vals_v[pl.ds(g * 16, 16)]
        valid = (pos >= v_start) & (pos < v_end)
        lo = jnp.full((16,), hbase, jnp.int32)
        hi = hbase + jnp.full((16,), BAGS_PER_REG, jnp.int32)
        for _ in range(N_SEARCH):
          mid = (lo + hi) >> 1
          ov = plsc.load_gather(off_v, [mid])
          le = ov <= pos
          lo = jnp.where(le, mid, lo)
          hi = jnp.where(le, hi, mid)
        gidx = jnp.where(valid, vals, wid * 47 + g * 16 + lane)
        didx = jnp.where(valid, srow + lo - hbase,
                         srow + DUMMY_OFF + (lane & 7))
        gidx_v[pl.ds(g * 16, 16)] = gidx
        didx_v[pl.ds(g * 16, 16)] = didx
        return carry2

      lax.fori_loop(0, N_VECS, vec_body, 0)
      pltpu.async_copy(tables_hbm.at[f].at[gidx_v], rows_v, sem).wait()
      pltpu.sync_copy(rows_v, acc.at[didx_v], add=True)
      return carry

    lax.fori_loop(0, n_chunks, chunk_body, 0)

    pltpu.sync_copy(acc.at[pl.ds(srow, BAGS_PER_REG)],
                    out_hbm.at[pl.ds(b0, BAGS_PER_REG), f])
    return carry0

  lax.fori_loop(0, N_REG, region_body, 0)


@jax.jit
def kernel(values, offsets, tables):
  values_pad = jnp.concatenate([values, jnp.zeros((K,), jnp.int32)])
  offsets_pad = jnp.concatenate(
      [offsets,
       jnp.full((OFF_PAD_LEN - (N_BAGS + 1),), TOTAL_VALUES, jnp.int32)])
  zeros = jnp.zeros((ACC_ROWS_PER_W, D), jnp.float32)

  mesh = plsc.VectorSubcoreMesh(core_axis_name="c", subcore_axis_name="s")
  run = pl.kernel(
      _body,
      out_type=jax.ShapeDtypeStruct((B, F_TABLES, D), jnp.float32),
      mesh=mesh,
      compiler_params=pltpu.CompilerParams(
          needs_layout_passes=False, use_tc_tiling_on_sc=False),
      scratch_types=[
          pltpu.VMEM((OFF_LOAD,), jnp.int32),
          pltpu.VMEM((K,), jnp.int32),
          pltpu.VMEM((K,), jnp.int32),
          pltpu.VMEM((K,), jnp.int32),
          pltpu.VMEM((K, D), jnp.float32),
          pltpu.VMEM((ACC_ROWS_PER_W, D), jnp.float32),
          pltpu.VMEM_SHARED((NS * ACC_ROWS_PER_W, D), jnp.float32),
          pltpu.SemaphoreType.DMA,
      ],
  )
  out = run(values_pad, offsets_pad, tables, zeros)
  return out.reshape(B, F_TABLES * D)


# speedup vs baseline: 1.0553x; 1.0553x over previous
import jax
import jax.numpy as jnp
from jax import lax
from jax.experimental import pallas as pl
from jax.experimental.pallas import tpu as pltpu
from jax.experimental.pallas import tpu_sc as plsc

F_TABLES = 26
B = 4096
V = 100000
D = 64
TOTAL_VALUES = 212992
N_BAGS = F_TABLES * B

NC = 2
NS = 16
NW = NC * NS

BAGS_PER_W = N_BAGS // NW
BAGS_PER_REG = 128
N_REG = BAGS_PER_W // BAGS_PER_REG
ACC_ROWS_PER_W = BAGS_PER_REG + 8
DUMMY_OFF = BAGS_PER_REG
N_SEARCH = 7

K = 256
N_VECS = K // 16

OFF_LOAD = 3344
OFF_PAD_LEN = (NW - 1) * BAGS_PER_W + OFF_LOAD


def _body(values_hbm, offsets_hbm, tables_hbm, zeros_hbm, out_hbm,
          off_v, vals_v, gidx_v, didx_v, rows_v, zeros_v, acc, sem):
  c = lax.axis_index("c")
  s = lax.axis_index("s")
  wid = c * NS + s
  bag_lo = wid * BAGS_PER_W
  srow = s * ACC_ROWS_PER_W

  lane = lax.iota(jnp.int32, 16)

  pltpu.sync_copy(offsets_hbm.at[pl.ds(bag_lo, OFF_LOAD)], off_v)
  pltpu.sync_copy(zeros_hbm, zeros_v)

  def off_scalar(idx16):
    v = off_v[pl.ds(idx16, 16)]
    return jnp.max(jnp.where(lane == 0, v, jnp.int32(-1)))

  def region_body(h, carry0):
    hbase = h * BAGS_PER_REG
    v_start = off_scalar(hbase)
    v_end = off_scalar(hbase + BAGS_PER_REG)
    f = lax.shift_right_arithmetic(bag_lo + hbase, 12)
    b0 = lax.bitwise_and(bag_lo + hbase, jnp.int32(B - 1))

    pltpu.sync_copy(zeros_v, acc.at[pl.ds(srow, ACC_ROWS_PER_W)])

    c0 = v_start & jnp.int32(~7)
    n_chunks = (v_end - c0 + jnp.int32(K - 1)) // jnp.int32(K)

    def chunk_body(ci, carry):
      base = pl.multiple_of(c0 + ci * jnp.int32(K), 8)
      pltpu.sync_copy(values_hbm.at[pl.ds(base, K)], vals_v)

      def vec_body(g, carry2):
        pos = base + g * 16 + lane
        vals = vals_v[pl.ds(g * 16, 16)]
        valid = (pos >= v_start) & (pos < v_end)
        lo = jnp.full((16,), hbase, jnp.int32)
        hi = hbase + jnp.full((16,), BAGS_PER_REG, jnp.int32)
        for _ in range(N_SEARCH):
          mid = (lo + hi) >> 1
          ov = plsc.load_gather(off_v, [mid])
          le = ov <= pos
          lo = jnp.where(le, mid, lo)
          hi = jnp.where(le, hi, mid)
        gidx = jnp.where(valid, vals, wid * 47 + g * 16 + lane)
        didx = jnp.where(valid, srow + lo - hbase,
                         srow + DUMMY_OFF + (lane & 7))
        gidx_v[pl.ds(g * 16, 16)] = gidx
        didx_v[pl.ds(g * 16, 16)] = didx
        return carry2

      lax.fori_loop(0, N_VECS, vec_body, 0)
      pltpu.async_copy(tables_hbm.at[f].at[gidx_v], rows_v, sem).wait()
      pltpu.sync_copy(rows_v, acc.at[didx_v], add=True)
      return carry

    lax.fori_loop(0, n_chunks, chunk_body, 0)

    pltpu.sync_copy(acc.at[pl.ds(srow, BAGS_PER_REG)],
                    out_hbm.at[pl.ds(b0, BAGS_PER_REG), f])
    return carry0

  lax.fori_loop(0, N_REG, region_body, 0)


@jax.jit
def kernel(values, offsets, tables):
  values_pad = jnp.concatenate([values, jnp.zeros((K,), jnp.int32)])
  offsets_pad = jnp.concatenate(
      [offsets,
       jnp.full((OFF_PAD_LEN - (N_BAGS + 1),), TOTAL_VALUES, jnp.int32)])
  zeros = jnp.zeros((ACC_ROWS_PER_W, D), jnp.float32)

  mesh = plsc.VectorSubcoreMesh(core_axis_name="c", subcore_axis_name="s")
  run = pl.kernel(
      _body,
      out_type=jax.ShapeDtypeStruct((B, F_TABLES, D), jnp.float32),
      mesh=mesh,
      compiler_params=pltpu.CompilerParams(
          needs_layout_passes=False, use_tc_tiling_on_sc=False),
      scratch_types=[
          pltpu.VMEM((OFF_LOAD,), jnp.int32),
          pltpu.VMEM((K,), jnp.int32),
          pltpu.VMEM((K,), jnp.int32),
          pltpu.VMEM((K,), jnp.int32),
          pltpu.VMEM((K, D), jnp.float32),
          pltpu.VMEM((ACC_ROWS_PER_W, D), jnp.float32),
          pltpu.VMEM_SHARED((NS * ACC_ROWS_PER_W, D), jnp.float32),
          pltpu.SemaphoreType.DMA,
      ],
  )
  out = run(values_pad, offsets_pad, tables, zeros)
  return out.reshape(B, F_TABLES * D)
